# Initial kernel scaffold; baseline (speedup 1.0000x reference)
import jax
import jax.numpy as jnp
from jax.experimental import pallas as pl


def kernel(_input, first_mat, second_mat):
    raise NotImplementedError("write your pallas kernel here")


# speedup vs baseline: 2.6023x; 2.6023x over previous
import functools
import math

import jax
import jax.numpy as jnp
from jax import lax
from jax.experimental import pallas as pl
from jax.experimental.pallas import tpu as pltpu
from jax.experimental.pallas import tpu_sc as plsc

_K = 4
_SECOND_K = 4
_NIDX = _K + _SECOND_K
_BATCH = 16384
_LEVEL_SQ_SUM = math.exp(math.log(1e24) / _NIDX)
_SCALE_NUM = _LEVEL_SQ_SUM ** (_NIDX / 2.0)

_NC, _NS, _L = 2, 16, 16
_NW = _NC * _NS
_RPW = _BATCH // _NW


def _gather_prod_sc(idx_t, first_mat, second_mat):
    mesh = plsc.VectorSubcoreMesh(core_axis_name="c", subcore_axis_name="s")

    @functools.partial(
        pl.kernel,
        out_type=jax.ShapeDtypeStruct((_BATCH,), jnp.float32),
        mesh=mesh,
        scratch_types=[
            [pltpu.VMEM((_RPW,), jnp.int32) for _ in range(_NIDX)],
            [pltpu.VMEM((_RPW,), jnp.float32) for _ in range(_NIDX)],
            pltpu.VMEM((_RPW,), jnp.float32),
            pltpu.SemaphoreType.DMA,
        ],
    )
    def k(idx_hbm, fm_hbm, sm_hbm, out_hbm, idxvs, valsvs, prodv, sem):
        wid = lax.axis_index("s") * _NC + lax.axis_index("c")
        base = wid * _RPW
        copies = []
        for j in range(_NIDX):
            pltpu.sync_copy(idx_hbm.at[j, pl.ds(base, _RPW)], idxvs[j])
            tbl = fm_hbm if j < _K else sm_hbm
            copies.append(pltpu.async_copy(tbl.at[idxvs[j]], valsvs[j], sem))
        for cp in copies:
            cp.wait()
        for i in range(_RPW // _L):
            sl = pl.ds(i * _L, _L)
            p = valsvs[0][sl]
            for j in range(1, _NIDX):
                p = p * valsvs[j][sl]
            prodv[sl] = p
        pltpu.sync_copy(prodv, out_hbm.at[pl.ds(base, _RPW)])

    return k(idx_t, first_mat, second_mat)


def _scale_tc(fm2d, sm2d, g2d):
    def body(fm_ref, sm_ref, g_ref, out_ref):
        fm = fm_ref[...]
        sm = sm_ref[...]
        s1 = jnp.sum(fm * fm)
        s2 = jnp.sum(sm * sm)
        scale = _SCALE_NUM / ((s1 * s1) * (s2 * s2))
        out_ref[...] = g_ref[...] * scale

    return pl.pallas_call(
        body,
        out_shape=jax.ShapeDtypeStruct(g2d.shape, jnp.float32),
    )(fm2d, sm2d, g2d)


def kernel(_input, first_mat, second_mat):
    idx_t = _input.astype(jnp.int32).T
    g = _gather_prod_sc(idx_t, first_mat, second_mat)
    out2 = _scale_tc(first_mat.reshape(1000, 1000),
                     second_mat.reshape(1000, 1000),
                     g.reshape(128, 128))
    return out2.reshape(_BATCH)
